# Initial kernel scaffold; baseline (speedup 1.0000x reference)
import jax
import jax.numpy as jnp
from jax.experimental import pallas as pl


def kernel(imu_seq, W1, b1, W2, b2, gate):
    raise NotImplementedError("write your pallas kernel here")


# speedup vs baseline: 6.0151x; 6.0151x over previous
import functools
import math

import jax
import jax.numpy as jnp
from jax.experimental import pallas as pl
from jax.experimental.pallas import tpu as pltpu

B, T, DIN, DH, DM, K = 16, 4096, 32, 64, 128, 32
SEG = T // K
TBLK = 1024
KBLK = TBLK // SEG


def _fused_kernel(x_ref, w1_ref, b1_ref, w2_ref, b2_ref, g_ref, out_ref):
    x = x_ref[0]
    h = jnp.dot(x, w1_ref[...], preferred_element_type=jnp.float32) + b1_ref[...]
    h = 0.5 * h * (1.0 + jax.lax.erf(h * (1.0 / math.sqrt(2.0))))
    pooled = h.reshape(KBLK, SEG, DH).sum(axis=1) * (1.0 / SEG)
    out = jnp.dot(pooled, w2_ref[...], preferred_element_type=jnp.float32)
    out_ref[0] = jnp.tanh(g_ref[0, 0]) * (out + b2_ref[...])


@jax.jit
def kernel(imu_seq, W1, b1, W2, b2, gate):
    grid = (B, T // TBLK)
    out = pl.pallas_call(
        _fused_kernel,
        grid=grid,
        in_specs=[
            pl.BlockSpec((1, TBLK, DIN), lambda b, j: (b, j, 0)),
            pl.BlockSpec((DIN, DH), lambda b, j: (0, 0)),
            pl.BlockSpec((1, DH), lambda b, j: (0, 0)),
            pl.BlockSpec((DH, DM), lambda b, j: (0, 0)),
            pl.BlockSpec((1, DM), lambda b, j: (0, 0)),
            pl.BlockSpec((1, 1), lambda b, j: (0, 0)),
        ],
        out_specs=pl.BlockSpec((1, KBLK, DM), lambda b, j: (b, j, 0)),
        out_shape=jax.ShapeDtypeStruct((B, K, DM), jnp.float32),
        compiler_params=pltpu.CompilerParams(
            dimension_semantics=("parallel", "parallel"),
        ),
    )(
        imu_seq,
        W1,
        b1.reshape(1, DH),
        W2,
        b2.reshape(1, DM),
        gate.reshape(1, 1),
    )
    return out


# speedup vs baseline: 9.1901x; 1.5278x over previous
import functools
import math

import jax
import jax.numpy as jnp
import numpy as np
from jax.experimental import pallas as pl
from jax.experimental.pallas import tpu as pltpu

B, T, DIN, DH, DM, K = 16, 4096, 32, 64, 128, 32
SEG = T // K
TBLK = 4096
KBLK = (K * TBLK) // T

_P_NP = np.kron(np.eye(KBLK, dtype=np.float32),
                np.full((1, SEG), 1.0 / SEG, dtype=np.float32))


def _fused_kernel(x_ref, w1_ref, b1_ref, p_ref, w2_ref, b2_ref, out_ref):
    x = x_ref[0]
    h = jnp.dot(x, w1_ref[...], preferred_element_type=jnp.float32) + b1_ref[...]
    h = 0.5 * h * (1.0 + jax.lax.erf(h * (1.0 / math.sqrt(2.0))))
    pooled = jnp.dot(p_ref[...], h, preferred_element_type=jnp.float32)
    out = jnp.dot(pooled, w2_ref[...], preferred_element_type=jnp.float32)
    out_ref[0] = out + b2_ref[...]


@jax.jit
def kernel(imu_seq, W1, b1, W2, b2, gate):
    g = jnp.tanh(gate)
    W2g = W2 * g
    b2g = (b2 * g).reshape(1, DM)
    P = jnp.asarray(_P_NP)
    grid = (B, T // TBLK)
    out = pl.pallas_call(
        _fused_kernel,
        grid=grid,
        in_specs=[
            pl.BlockSpec((1, TBLK, DIN), lambda b, j: (b, j, 0)),
            pl.BlockSpec((DIN, DH), lambda b, j: (0, 0)),
            pl.BlockSpec((1, DH), lambda b, j: (0, 0)),
            pl.BlockSpec((KBLK, TBLK), lambda b, j: (0, 0)),
            pl.BlockSpec((DH, DM), lambda b, j: (0, 0)),
            pl.BlockSpec((1, DM), lambda b, j: (0, 0)),
        ],
        out_specs=pl.BlockSpec((1, KBLK, DM), lambda b, j: (b, j, 0)),
        out_shape=jax.ShapeDtypeStruct((B, K, DM), jnp.float32),
        compiler_params=pltpu.CompilerParams(
            dimension_semantics=("parallel", "parallel"),
        ),
    )(
        imu_seq,
        W1,
        b1.reshape(1, DH),
        P,
        W2g,
        b2g,
    )
    return out


# speedup vs baseline: 9.7473x; 1.0606x over previous
import functools
import math

import jax
import jax.numpy as jnp
import numpy as np
from jax.experimental import pallas as pl
from jax.experimental.pallas import tpu as pltpu

B, T, DIN, DH, DM, K = 16, 4096, 32, 64, 128, 32
SEG = T // K
TBLK = 4096
KBLK = (K * TBLK) // T

_P_NP = np.kron(np.eye(KBLK, dtype=np.float32),
                np.full((1, SEG), 1.0 / SEG, dtype=np.float32))


def _fused_kernel(x_ref, w1_ref, b1_ref, p_ref, w2_ref, b2_ref, g_ref, out_ref):
    x = x_ref[0]
    h = jnp.dot(x, w1_ref[...], preferred_element_type=jnp.float32) + b1_ref[...]
    h = 0.5 * h * (1.0 + jax.lax.erf(h * (1.0 / math.sqrt(2.0))))
    pooled = jnp.dot(p_ref[...], h, preferred_element_type=jnp.float32)
    out = jnp.dot(pooled, w2_ref[...], preferred_element_type=jnp.float32)
    g = jnp.tanh(g_ref[0, 0])
    out_ref[0] = g * (out + b2_ref[...])


@jax.jit
def kernel(imu_seq, W1, b1, W2, b2, gate):
    P = jnp.asarray(_P_NP)
    grid = (B, T // TBLK)
    out = pl.pallas_call(
        _fused_kernel,
        grid=grid,
        in_specs=[
            pl.BlockSpec((1, TBLK, DIN), lambda b, j: (b, j, 0)),
            pl.BlockSpec((DIN, DH), lambda b, j: (0, 0)),
            pl.BlockSpec((1, DH), lambda b, j: (0, 0)),
            pl.BlockSpec((KBLK, TBLK), lambda b, j: (0, 0)),
            pl.BlockSpec((DH, DM), lambda b, j: (0, 0)),
            pl.BlockSpec((1, DM), lambda b, j: (0, 0)),
            pl.BlockSpec((1, 1), lambda b, j: (0, 0)),
        ],
        out_specs=pl.BlockSpec((1, KBLK, DM), lambda b, j: (b, j, 0)),
        out_shape=jax.ShapeDtypeStruct((B, K, DM), jnp.float32),
        compiler_params=pltpu.CompilerParams(
            dimension_semantics=("parallel", "parallel"),
        ),
    )(
        imu_seq,
        W1,
        b1.reshape(1, DH),
        P,
        W2,
        b2.reshape(1, DM),
        gate.reshape(1, 1),
    )
    return out
